# baseline (speedup 1.0000x reference)
import jax
import jax.numpy as jnp
from jax.experimental import pallas as pl

BATCH_PER_STEP = 2
K_CHUNK = 256


def _vq_kernel(z_ref, cb_ref, zq_ref, idx_ref):
    cb = cb_ref[...]
    K = cb.shape[0]
    cbm2 = cb * -2.0
    cb2 = jnp.sum(cb * cb, axis=1, keepdims=True)

    for j in range(BATCH_PER_STEP):
        zb = z_ref[j]
        N = zb.shape[1]
        z2 = jnp.sum(zb * zb, axis=0, keepdims=True)

        m_run = jnp.full((1, N), jnp.inf, dtype=jnp.float32)
        idx_run = jnp.zeros((1, N), dtype=jnp.int32)
        for kc in range(K // K_CHUNK):
            lo = kc * K_CHUNK
            dchunk = jax.lax.dot_general(
                cbm2[lo:lo + K_CHUNK], zb, (((1,), (0,)), ((), ())),
                preferred_element_type=jnp.float32,
            )
            dchunk = dchunk + z2
            dchunk = dchunk + cb2[lo:lo + K_CHUNK]
            kiota = lo + jax.lax.broadcasted_iota(jnp.int32, dchunk.shape, 0)
            mc = jnp.min(dchunk, axis=0, keepdims=True)
            ic = jnp.min(jnp.where(dchunk == mc, kiota, K), axis=0,
                         keepdims=True)
            upd = mc < m_run
            idx_run = jnp.where(upd, ic, idx_run)
            m_run = jnp.where(upd, mc, m_run)

        idx = idx_run[0]
        idx_ref[j, 0] = idx

        kiota_full = jax.lax.broadcasted_iota(jnp.int32, (K, N), 0)
        onehot = (kiota_full == idx[None, :]).astype(jnp.bfloat16)
        zq = jax.lax.dot_general(
            cb, onehot, (((0,), (0,)), ((), ())),
            preferred_element_type=jnp.float32,
        )
        zq_ref[j] = zq


def kernel(z, codebook):
    B, C, H, W = z.shape
    K, _ = codebook.shape
    N = H * W
    NB = B // BATCH_PER_STEP

    z3 = z.reshape(B, C, N)

    zq3, idx3 = pl.pallas_call(
        _vq_kernel,
        grid=(NB,),
        in_specs=[
            pl.BlockSpec((BATCH_PER_STEP, C, N), lambda b: (b, 0, 0)),
            pl.BlockSpec((K, C), lambda b: (0, 0)),
        ],
        out_specs=[
            pl.BlockSpec((BATCH_PER_STEP, C, N), lambda b: (b, 0, 0)),
            pl.BlockSpec((BATCH_PER_STEP, 1, N), lambda b: (b, 0, 0)),
        ],
        out_shape=[
            jax.ShapeDtypeStruct((B, C, N), jnp.float32),
            jax.ShapeDtypeStruct((B, 1, N), jnp.int32),
        ],
    )(z3, codebook)

    zq = zq3.reshape(B, C, H, W)
    idx = idx3.reshape(B, H, W)
    return zq, idx


# speedup vs baseline: 1.1255x; 1.1255x over previous
import jax
import jax.numpy as jnp
from jax.experimental import pallas as pl

BATCH_PER_STEP = 2


def _vq_kernel(z_ref, cb_ref, zq_ref, idx_ref):
    cb = cb_ref[...]
    K = cb.shape[0]
    cbm2 = cb * -2.0
    cb2 = jnp.sum(cb * cb, axis=1, keepdims=True)

    for j in range(BATCH_PER_STEP):
        zb = z_ref[j]
        N = zb.shape[1]
        dist = jax.lax.dot_general(
            cbm2, zb, (((1,), (0,)), ((), ())),
            preferred_element_type=jnp.float32,
        )
        z2 = jnp.sum(zb * zb, axis=0, keepdims=True)
        dist = dist + z2
        dist = dist + cb2

        idx = jnp.argmin(dist, axis=0).astype(jnp.int32)
        idx_ref[j, 0] = idx

        kiota = jax.lax.broadcasted_iota(jnp.int32, (K, N), 0)
        onehot = (kiota == idx[None, :]).astype(jnp.bfloat16)
        zq = jax.lax.dot_general(
            cb, onehot, (((0,), (0,)), ((), ())),
            preferred_element_type=jnp.float32,
        )
        zq_ref[j] = zq


def kernel(z, codebook):
    B, C, H, W = z.shape
    K, _ = codebook.shape
    N = H * W
    NB = B // BATCH_PER_STEP

    z3 = z.reshape(B, C, N)

    zq3, idx3 = pl.pallas_call(
        _vq_kernel,
        grid=(NB,),
        in_specs=[
            pl.BlockSpec((BATCH_PER_STEP, C, N), lambda b: (b, 0, 0)),
            pl.BlockSpec((K, C), lambda b: (0, 0)),
        ],
        out_specs=[
            pl.BlockSpec((BATCH_PER_STEP, C, N), lambda b: (b, 0, 0)),
            pl.BlockSpec((BATCH_PER_STEP, 1, N), lambda b: (b, 0, 0)),
        ],
        out_shape=[
            jax.ShapeDtypeStruct((B, C, N), jnp.float32),
            jax.ShapeDtypeStruct((B, 1, N), jnp.int32),
        ],
    )(z3, codebook)

    zq = zq3.reshape(B, C, H, W)
    idx = idx3.reshape(B, H, W)
    return zq, idx
